# baseline (speedup 1.0000x reference)
import functools

import jax
import jax.numpy as jnp
from jax import lax
from jax.experimental import pallas as pl
from jax.experimental.pallas import tpu as pltpu
from jax.experimental.pallas import tpu_sc as plsc


def _sc_gather(confidence, index):
    n_rows, n_cols = confidence.shape
    b = index.shape[0]
    info = plsc.get_sparse_core_info()
    nw = info.num_cores * info.num_subcores
    b_per_w = b // nw
    mesh = plsc.VectorSubcoreMesh(core_axis_name="c", subcore_axis_name="s")

    @functools.partial(
        pl.kernel,
        mesh=mesh,
        out_type=jax.ShapeDtypeStruct((b, n_cols), jnp.float32),
        scratch_types=[
            pltpu.VMEM((b_per_w,), jnp.int32),
            pltpu.VMEM((b_per_w, n_cols), jnp.float32),
            pltpu.SemaphoreType.DMA,
        ],
        compiler_params=pltpu.CompilerParams(use_tc_tiling_on_sc=False),
    )
    def gather_kernel(table_hbm, idx_hbm, out_hbm, idx_v, rows_v, sem):
        wid = lax.axis_index("s") * info.num_cores + lax.axis_index("c")
        base = wid * b_per_w
        pltpu.sync_copy(idx_hbm.at[pl.ds(base, b_per_w)], idx_v)
        pltpu.async_copy(table_hbm.at[idx_v], rows_v, sem).wait()
        pltpu.sync_copy(rows_v, out_hbm.at[pl.ds(base, b_per_w)])

    return gather_kernel(confidence, index)


def _loss_body(ow_ref, os_ref, tg_ref, acc_ref):
    i = pl.program_id(0)
    ow = ow_ref[...]
    osl = os_ref[...]
    c = ow_ref.shape[1]
    tg = tg_ref[...]
    t = tg[:, :c]

    mw = jnp.max(ow, axis=1, keepdims=True)
    ew = jnp.exp(ow - mw)
    sw = jnp.sum(ew, axis=1, keepdims=True)
    lse_w = mw + jnp.log(sw)

    ms = jnp.max(osl, axis=1, keepdims=True)
    es = jnp.exp(osl - ms)
    ss = jnp.sum(es, axis=1, keepdims=True)
    lse_s = ms + jnp.log(ss)

    pos = t > 0.0
    safe_t = jnp.where(pos, t, 1.0)
    xlogy = t * jnp.log(safe_t)
    tsum = jnp.sum(t, axis=1, keepdims=True)

    part = (2.0 * jnp.sum(xlogy)
            - jnp.sum(t * ow) - jnp.sum(t * osl)
            + jnp.sum(tsum * (lse_w + lse_s)))

    @pl.when(i == 0)
    def _():
        acc_ref[0, 0] = 0.0

    acc_ref[0, 0] += part

    any_zero = jnp.sum((t == 0.0).astype(jnp.float32)) > 0.0

    @pl.when(any_zero)
    def _():
        pred_w = ew / sw
        pred_s = es / ss
        neg = (t == 0.0).astype(jnp.float32)
        sup = neg * (-jnp.log(jnp.abs(1.0 - pred_w) + 1e-9)
                     - jnp.log(jnp.abs(1.0 - pred_s) + 1e-9))
        acc_ref[0, 0] += jnp.sum(sup)


def _loss_tc(output_w, output_s, target, block_rows=256, interpret=False):
    b, c = output_w.shape
    ct = target.shape[1]
    grid = b // block_rows
    acc = pl.pallas_call(
        _loss_body,
        grid=(grid,),
        in_specs=[
            pl.BlockSpec((block_rows, c), lambda i: (i, 0)),
            pl.BlockSpec((block_rows, c), lambda i: (i, 0)),
            pl.BlockSpec((block_rows, ct), lambda i: (i, 0)),
        ],
        out_specs=pl.BlockSpec((1, 1), lambda i: (0, 0),
                               memory_space=pltpu.SMEM),
        out_shape=jax.ShapeDtypeStruct((1, 1), jnp.float32),
        interpret=interpret,
    )(output_w, output_s, target)
    return acc[0, 0] / b


_NQ = 8


def _sc_gather_head(confidence, index, head=896):
    b = index.shape[0]
    info = plsc.get_sparse_core_info()
    nw = info.num_cores * info.num_subcores
    b_per_w = b // nw
    mesh = plsc.VectorSubcoreMesh(core_axis_name="c", subcore_axis_name="s")

    @functools.partial(
        pl.kernel,
        mesh=mesh,
        out_type=jax.ShapeDtypeStruct((b, head), jnp.float32),
        scratch_types=[
            pltpu.VMEM((b_per_w,), jnp.int32),
            pltpu.VMEM((b_per_w, head), jnp.float32),
            pltpu.SemaphoreType.DMA,
        ],
    )
    def gather_kernel(table_hbm, idx_hbm, out_hbm, idx_v, rows_v, sem):
        wid = lax.axis_index("s") * info.num_cores + lax.axis_index("c")
        base = wid * b_per_w
        pltpu.sync_copy(idx_hbm.at[pl.ds(base, b_per_w)], idx_v)
        pltpu.async_copy(
            table_hbm.at[idx_v, pl.ds(0, head)], rows_v, sem).wait()
        pltpu.sync_copy(rows_v, out_hbm.at[pl.ds(base, b_per_w)])

    return gather_kernel(confidence, index)


def _rne_bf16_hi(x_i32):
    rnd = jnp.bitwise_and(lax.shift_right_logical(x_i32, 16), 1) + 0x7FFF
    return jnp.bitwise_and(x_i32 + rnd, jnp.int32(-65536))


def _sc_gather_head_bf16(confidence, index, head=896):
    b = index.shape[0]
    half = 512
    info = plsc.get_sparse_core_info()
    nw = info.num_cores * info.num_subcores
    b_per_w = b // nw
    mesh = plsc.VectorSubcoreMesh(core_axis_name="c", subcore_axis_name="s")
    lanes = info.num_lanes

    chunk = 64
    n_chunks = b_per_w // chunk

    @functools.partial(
        pl.kernel,
        mesh=mesh,
        out_type=jax.ShapeDtypeStruct((b, half), jnp.int32),
        scratch_types=[
            pltpu.VMEM((b_per_w,), jnp.int32),
            pltpu.VMEM((chunk, head), jnp.float32),
            pltpu.VMEM((chunk, half), jnp.int32),
            pltpu.SemaphoreType.DMA,
        ],
        compiler_params=pltpu.CompilerParams(needs_layout_passes=False),
    )
    def gather_kernel(table_hbm, idx_hbm, out_hbm, idx_v, rows_v, pk_v, sem):
        wid = lax.axis_index("s") * info.num_cores + lax.axis_index("c")
        base = wid * b_per_w
        pltpu.sync_copy(idx_hbm.at[pl.ds(base, b_per_w)], idx_v)
        for ci in range(n_chunks):
            idxc = idx_v.at[pl.ds(ci * chunk, chunk)]
            pltpu.async_copy(
                table_hbm.at[idxc, pl.ds(0, head)], rows_v, sem).wait()

            def row(r, _):
                for j in range(half // lanes):
                    lo_f = rows_v[r, pl.ds(j * lanes, lanes)]
                    if (j + 1) * lanes <= head - half:
                        hi_f = rows_v[r, pl.ds(half + j * lanes, lanes)]
                    else:
                        hi_f = jnp.zeros((lanes,), jnp.float32)
                    pk = plsc.pack(lo_f, hi_f,
                                   format=plsc.PackFormat.INTERLEAVED)
                    pk_v[r, pl.ds(j * lanes, lanes)] = plsc.bitcast(
                        pk, jnp.int32)
                return 0

            lax.fori_loop(0, chunk, row, 0)
            pltpu.sync_copy(
                pk_v, out_hbm.at[pl.ds(base + ci * chunk, chunk)])

    return gather_kernel(confidence, index)


def _sc_gather_split(confidence, index):
    n_rows, n_cols = confidence.shape
    head = 896
    cpad = 1024
    b = index.shape[0]
    info = plsc.get_sparse_core_info()
    nw = info.num_cores * info.num_subcores
    b_per_w = b // nw
    chunk = 64
    n_chunks = b_per_w // chunk
    mesh = plsc.VectorSubcoreMesh(core_axis_name="c", subcore_axis_name="s")

    @functools.partial(
        pl.kernel,
        mesh=mesh,
        out_type=jax.ShapeDtypeStruct((b, cpad), jnp.float32),
        scratch_types=[
            pltpu.VMEM((b_per_w,), jnp.int32),
            pltpu.VMEM((chunk, cpad), jnp.float32),
            pltpu.SemaphoreType.DMA,
            pltpu.SemaphoreType.DMA,
        ],
    )
    def gather_kernel(table_hbm, idx_hbm, out_hbm, idx_v, rows_v,
                      sem_h, sem_t):
        wid = lax.axis_index("s") * info.num_cores + lax.axis_index("c")
        base = wid * b_per_w
        pltpu.sync_copy(idx_hbm.at[pl.ds(base, b_per_w)], idx_v)
        for ci in range(n_chunks):
            idxc = idx_v.at[pl.ds(ci * chunk, chunk)]
            ch = pltpu.async_copy(
                table_hbm.at[idxc, pl.ds(0, head)],
                rows_v.at[:, pl.ds(0, head)], sem_h)
            ct = pltpu.async_copy(
                table_hbm.at[idxc, pl.ds(head, cpad - head)],
                rows_v.at[:, pl.ds(head, cpad - head)], sem_t)
            ch.wait()
            ct.wait()
            pltpu.sync_copy(rows_v, out_hbm.at[pl.ds(base + ci * chunk, chunk)])

    return gather_kernel(confidence, index)


def _loss_tail_body(idx_ref, ow_ref, os_ref, hd_ref, conf_ref, acc_ref,
                    tail, sem, row_offset=0):
    i = pl.program_id(0)
    ngrid = pl.num_programs(0)
    block_rows = ow_ref.shape[0]
    c = ow_ref.shape[1]
    half = hd_ref.shape[1]
    head = 896
    tw = c - head

    def issue(step, slot):
        base = row_offset + step * block_rows

        def one(j, _):
            jj = j * _NQ
            for k in range(_NQ):
                r = idx_ref[base + jj + k]
                pltpu.make_async_copy(
                    conf_ref.at[pl.ds(r, 1), pl.ds(head, tw)],
                    tail.at[slot, pl.ds(jj + k, 1), :],
                    sem.at[slot, k],
                ).start()
            return 0

        lax.fori_loop(0, block_rows // _NQ, one, 0)

    @pl.when(i == 0)
    def _():
        issue(0, 0)

    @pl.when(i + 1 < ngrid)
    def _():
        issue(i + 1, (i + 1) % 2)

    slot = i % 2
    for k in range(_NQ):
        pltpu.make_async_copy(
            conf_ref.at[pl.ds(0, block_rows // _NQ), pl.ds(head, tw)],
            tail.at[slot, pl.ds(0, block_rows // _NQ), :],
            sem.at[slot, k],
        ).wait()

    ow = ow_ref[...]
    osl = os_ref[...]
    pk = hd_ref[...]
    lo = lax.bitcast_convert_type(lax.shift_left(pk, 16), jnp.float32)
    hi = lax.bitcast_convert_type(
        jnp.bitwise_and(pk, jnp.int32(-65536)), jnp.float32)
    t = jnp.concatenate([lo, hi[:, :head - half], tail[slot]], axis=1)

    mw = jnp.max(ow, axis=1, keepdims=True)
    ew = jnp.exp(ow - mw)
    sw = jnp.sum(ew, axis=1, keepdims=True)
    lse_w = mw + jnp.log(sw)

    ms = jnp.max(osl, axis=1, keepdims=True)
    es = jnp.exp(osl - ms)
    ss = jnp.sum(es, axis=1, keepdims=True)
    lse_s = ms + jnp.log(ss)

    pos = t > 0.0
    safe_t = jnp.where(pos, t, 1.0)
    xlogy = t * jnp.log(safe_t)
    tsum = jnp.sum(t, axis=1, keepdims=True)

    part = (2.0 * jnp.sum(xlogy)
            - jnp.sum(t * ow) - jnp.sum(t * osl)
            + jnp.sum(tsum * (lse_w + lse_s)))

    @pl.when(i == 0)
    def _():
        acc_ref[0, 0] = 0.0

    acc_ref[0, 0] += part

    any_zero = jnp.sum((t == 0.0).astype(jnp.float32)) > 0.0

    @pl.when(any_zero)
    def _():
        pred_w = ew / sw
        pred_s = es / ss
        neg = (t == 0.0).astype(jnp.float32)
        sup = neg * (-jnp.log(jnp.abs(1.0 - pred_w) + 1e-9)
                     - jnp.log(jnp.abs(1.0 - pred_s) + 1e-9))
        acc_ref[0, 0] += jnp.sum(sup)


def _loss_tail_tc(output_w, output_s, head_tgt, confidence, index,
                  block_rows=256, row_offset=0, rows=None):
    b, c = output_w.shape
    if rows is None:
        rows = b
    half = head_tgt.shape[1]
    head = 896
    grid = rows // block_rows
    blk_off = row_offset // block_rows
    grid_spec = pltpu.PrefetchScalarGridSpec(
        num_scalar_prefetch=1,
        grid=(grid,),
        in_specs=[
            pl.BlockSpec((block_rows, c), lambda i, idx: (blk_off + i, 0)),
            pl.BlockSpec((block_rows, c), lambda i, idx: (blk_off + i, 0)),
            pl.BlockSpec((block_rows, half), lambda i, idx: (i, 0)),
            pl.BlockSpec(memory_space=pl.ANY),
        ],
        out_specs=pl.BlockSpec((1, 1), lambda i, idx: (0, 0),
                               memory_space=pltpu.SMEM),
        scratch_shapes=[
            pltpu.VMEM((2, block_rows, c - head), jnp.float32),
            pltpu.SemaphoreType.DMA((2, _NQ)),
        ],
    )
    acc = pl.pallas_call(
        functools.partial(_loss_tail_body, row_offset=row_offset),
        grid_spec=grid_spec,
        out_shape=jax.ShapeDtypeStruct((1, 1), jnp.float32),
    )(index, output_w, output_s, head_tgt, confidence)
    return acc[0, 0]


def _fused_body(idx_ref, ow_ref, os_ref, conf_ref, acc_ref, tgt, sem):
    i = pl.program_id(0)
    ngrid = pl.num_programs(0)
    block_rows = ow_ref.shape[0]
    c = ow_ref.shape[1]

    def issue(step, slot):
        base = step * block_rows

        def one(j, _):
            jj = j * _NQ
            for k in range(_NQ):
                r = idx_ref[base + jj + k]
                pltpu.make_async_copy(
                    conf_ref.at[pl.ds(r, 1), :],
                    tgt.at[slot, pl.ds(jj + k, 1), :],
                    sem.at[slot, k],
                ).start()
            return 0

        lax.fori_loop(0, block_rows // _NQ, one, 0)

    @pl.when(i == 0)
    def _():
        issue(0, 0)

    @pl.when(i + 1 < ngrid)
    def _():
        issue(i + 1, (i + 1) % 2)

    slot = i % 2
    for k in range(_NQ):
        pltpu.make_async_copy(
            conf_ref.at[pl.ds(0, block_rows // _NQ), :],
            tgt.at[slot, pl.ds(0, block_rows // _NQ), :],
            sem.at[slot, k],
        ).wait()

    ow = ow_ref[...]
    osl = os_ref[...]
    t = tgt[slot]

    mw = jnp.max(ow, axis=1, keepdims=True)
    ew = jnp.exp(ow - mw)
    sw = jnp.sum(ew, axis=1, keepdims=True)
    lse_w = mw + jnp.log(sw)

    ms = jnp.max(osl, axis=1, keepdims=True)
    es = jnp.exp(osl - ms)
    ss = jnp.sum(es, axis=1, keepdims=True)
    lse_s = ms + jnp.log(ss)

    pos = t > 0.0
    safe_t = jnp.where(pos, t, 1.0)
    xlogy = t * jnp.log(safe_t)
    tsum = jnp.sum(t, axis=1, keepdims=True)

    part = (2.0 * jnp.sum(xlogy)
            - jnp.sum(t * ow) - jnp.sum(t * osl)
            + jnp.sum(tsum * (lse_w + lse_s)))

    @pl.when(i == 0)
    def _():
        acc_ref[0, 0] = 0.0

    acc_ref[0, 0] += part

    any_zero = jnp.sum((t == 0.0).astype(jnp.float32)) > 0.0

    @pl.when(any_zero)
    def _():
        pred_w = ew / sw
        pred_s = es / ss
        neg = (t == 0.0).astype(jnp.float32)
        sup = neg * (-jnp.log(jnp.abs(1.0 - pred_w) + 1e-9)
                     - jnp.log(jnp.abs(1.0 - pred_s) + 1e-9))
        acc_ref[0, 0] += jnp.sum(sup)


def _fused_tc(output_w, output_s, confidence, index, block_rows=256):
    b, c = output_w.shape
    grid = b // block_rows
    grid_spec = pltpu.PrefetchScalarGridSpec(
        num_scalar_prefetch=1,
        grid=(grid,),
        in_specs=[
            pl.BlockSpec((block_rows, c), lambda i, idx: (i, 0)),
            pl.BlockSpec((block_rows, c), lambda i, idx: (i, 0)),
            pl.BlockSpec(memory_space=pl.ANY),
        ],
        out_specs=pl.BlockSpec((1, 1), lambda i, idx: (0, 0),
                               memory_space=pltpu.SMEM),
        scratch_shapes=[
            pltpu.VMEM((2, block_rows, c), jnp.float32),
            pltpu.SemaphoreType.DMA((2, _NQ)),
        ],
    )
    acc = pl.pallas_call(
        _fused_body,
        grid_spec=grid_spec,
        out_shape=jax.ShapeDtypeStruct((1, 1), jnp.float32),
    )(index, output_w, output_s, confidence)
    return acc[0, 0] / b


def kernel(output_w, output_s, feat_w, feat_s, confidence, index):
    del feat_w, feat_s
    b = index.shape[0]
    hb = b // 2
    head_tgt1 = _sc_gather_head_bf16(confidence, index[:hb])
    head_tgt2 = _sc_gather_head_bf16(confidence, index[hb:])
    acc1 = _loss_tail_tc(output_w, output_s, head_tgt1, confidence, index,
                         block_rows=512, row_offset=0, rows=hb)
    acc2 = _loss_tail_tc(output_w, output_s, head_tgt2, confidence, index,
                         block_rows=512, row_offset=hb, rows=b - hb)
    return (acc1 + acc2) / b


# speedup vs baseline: 1.0243x; 1.0243x over previous
import functools

import jax
import jax.numpy as jnp
from jax import lax
from jax.experimental import pallas as pl
from jax.experimental.pallas import tpu as pltpu
from jax.experimental.pallas import tpu_sc as plsc


def _sc_gather(confidence, index):
    n_rows, n_cols = confidence.shape
    b = index.shape[0]
    info = plsc.get_sparse_core_info()
    nw = info.num_cores * info.num_subcores
    b_per_w = b // nw
    mesh = plsc.VectorSubcoreMesh(core_axis_name="c", subcore_axis_name="s")

    @functools.partial(
        pl.kernel,
        mesh=mesh,
        out_type=jax.ShapeDtypeStruct((b, n_cols), jnp.float32),
        scratch_types=[
            pltpu.VMEM((b_per_w,), jnp.int32),
            pltpu.VMEM((b_per_w, n_cols), jnp.float32),
            pltpu.SemaphoreType.DMA,
        ],
        compiler_params=pltpu.CompilerParams(use_tc_tiling_on_sc=False),
    )
    def gather_kernel(table_hbm, idx_hbm, out_hbm, idx_v, rows_v, sem):
        wid = lax.axis_index("s") * info.num_cores + lax.axis_index("c")
        base = wid * b_per_w
        pltpu.sync_copy(idx_hbm.at[pl.ds(base, b_per_w)], idx_v)
        pltpu.async_copy(table_hbm.at[idx_v], rows_v, sem).wait()
        pltpu.sync_copy(rows_v, out_hbm.at[pl.ds(base, b_per_w)])

    return gather_kernel(confidence, index)


def _loss_body(ow_ref, os_ref, tg_ref, acc_ref):
    i = pl.program_id(0)
    ow = ow_ref[...]
    osl = os_ref[...]
    c = ow_ref.shape[1]
    tg = tg_ref[...]
    t = tg[:, :c]

    mw = jnp.max(ow, axis=1, keepdims=True)
    ew = jnp.exp(ow - mw)
    sw = jnp.sum(ew, axis=1, keepdims=True)
    lse_w = mw + jnp.log(sw)

    ms = jnp.max(osl, axis=1, keepdims=True)
    es = jnp.exp(osl - ms)
    ss = jnp.sum(es, axis=1, keepdims=True)
    lse_s = ms + jnp.log(ss)

    pos = t > 0.0
    safe_t = jnp.where(pos, t, 1.0)
    xlogy = t * jnp.log(safe_t)
    tsum = jnp.sum(t, axis=1, keepdims=True)

    part = (2.0 * jnp.sum(xlogy)
            - jnp.sum(t * ow) - jnp.sum(t * osl)
            + jnp.sum(tsum * (lse_w + lse_s)))

    @pl.when(i == 0)
    def _():
        acc_ref[0, 0] = 0.0

    acc_ref[0, 0] += part

    any_zero = jnp.sum((t == 0.0).astype(jnp.float32)) > 0.0

    @pl.when(any_zero)
    def _():
        pred_w = ew / sw
        pred_s = es / ss
        neg = (t == 0.0).astype(jnp.float32)
        sup = neg * (-jnp.log(jnp.abs(1.0 - pred_w) + 1e-9)
                     - jnp.log(jnp.abs(1.0 - pred_s) + 1e-9))
        acc_ref[0, 0] += jnp.sum(sup)


def _loss_tc(output_w, output_s, target, block_rows=256, interpret=False):
    b, c = output_w.shape
    ct = target.shape[1]
    grid = b // block_rows
    acc = pl.pallas_call(
        _loss_body,
        grid=(grid,),
        in_specs=[
            pl.BlockSpec((block_rows, c), lambda i: (i, 0)),
            pl.BlockSpec((block_rows, c), lambda i: (i, 0)),
            pl.BlockSpec((block_rows, ct), lambda i: (i, 0)),
        ],
        out_specs=pl.BlockSpec((1, 1), lambda i: (0, 0),
                               memory_space=pltpu.SMEM),
        out_shape=jax.ShapeDtypeStruct((1, 1), jnp.float32),
        interpret=interpret,
    )(output_w, output_s, target)
    return acc[0, 0] / b


_NQ = 8


def _sc_gather_head(confidence, index, head=896):
    b = index.shape[0]
    info = plsc.get_sparse_core_info()
    nw = info.num_cores * info.num_subcores
    b_per_w = b // nw
    mesh = plsc.VectorSubcoreMesh(core_axis_name="c", subcore_axis_name="s")

    @functools.partial(
        pl.kernel,
        mesh=mesh,
        out_type=jax.ShapeDtypeStruct((b, head), jnp.float32),
        scratch_types=[
            pltpu.VMEM((b_per_w,), jnp.int32),
            pltpu.VMEM((b_per_w, head), jnp.float32),
            pltpu.SemaphoreType.DMA,
        ],
    )
    def gather_kernel(table_hbm, idx_hbm, out_hbm, idx_v, rows_v, sem):
        wid = lax.axis_index("s") * info.num_cores + lax.axis_index("c")
        base = wid * b_per_w
        pltpu.sync_copy(idx_hbm.at[pl.ds(base, b_per_w)], idx_v)
        pltpu.async_copy(
            table_hbm.at[idx_v, pl.ds(0, head)], rows_v, sem).wait()
        pltpu.sync_copy(rows_v, out_hbm.at[pl.ds(base, b_per_w)])

    return gather_kernel(confidence, index)


def _rne_bf16_hi(x_i32):
    rnd = jnp.bitwise_and(lax.shift_right_logical(x_i32, 16), 1) + 0x7FFF
    return jnp.bitwise_and(x_i32 + rnd, jnp.int32(-65536))


def _sc_gather_head_bf16(confidence, index, head=896):
    b = index.shape[0]
    half = 512
    info = plsc.get_sparse_core_info()
    nw = info.num_cores * info.num_subcores
    b_per_w = b // nw
    mesh = plsc.VectorSubcoreMesh(core_axis_name="c", subcore_axis_name="s")
    lanes = info.num_lanes

    chunk = 64
    n_chunks = b_per_w // chunk

    @functools.partial(
        pl.kernel,
        mesh=mesh,
        out_type=jax.ShapeDtypeStruct((b, half), jnp.int32),
        scratch_types=[
            pltpu.VMEM((b_per_w,), jnp.int32),
            pltpu.VMEM((chunk, head), jnp.float32),
            pltpu.VMEM((chunk, half), jnp.int32),
            pltpu.SemaphoreType.DMA,
        ],
        compiler_params=pltpu.CompilerParams(needs_layout_passes=False),
    )
    def gather_kernel(table_hbm, idx_hbm, out_hbm, idx_v, rows_v, pk_v, sem):
        wid = lax.axis_index("s") * info.num_cores + lax.axis_index("c")
        base = wid * b_per_w
        pltpu.sync_copy(idx_hbm.at[pl.ds(base, b_per_w)], idx_v)
        for ci in range(n_chunks):
            idxc = idx_v.at[pl.ds(ci * chunk, chunk)]
            pltpu.async_copy(
                table_hbm.at[idxc, pl.ds(0, head)], rows_v, sem).wait()

            def row(r, _):
                for j in range(half // lanes):
                    lo_f = rows_v[r, pl.ds(j * lanes, lanes)]
                    if (j + 1) * lanes <= head - half:
                        hi_f = rows_v[r, pl.ds(half + j * lanes, lanes)]
                    else:
                        hi_f = jnp.zeros((lanes,), jnp.float32)
                    pk = plsc.pack(lo_f, hi_f,
                                   format=plsc.PackFormat.INTERLEAVED)
                    pk_v[r, pl.ds(j * lanes, lanes)] = plsc.bitcast(
                        pk, jnp.int32)
                return 0

            lax.fori_loop(0, chunk, row, 0)
            pltpu.sync_copy(
                pk_v, out_hbm.at[pl.ds(base + ci * chunk, chunk)])

    return gather_kernel(confidence, index)


def _sc_gather_split(confidence, index):
    n_rows, n_cols = confidence.shape
    head = 896
    cpad = 1024
    b = index.shape[0]
    info = plsc.get_sparse_core_info()
    nw = info.num_cores * info.num_subcores
    b_per_w = b // nw
    chunk = 64
    n_chunks = b_per_w // chunk
    mesh = plsc.VectorSubcoreMesh(core_axis_name="c", subcore_axis_name="s")

    @functools.partial(
        pl.kernel,
        mesh=mesh,
        out_type=jax.ShapeDtypeStruct((b, cpad), jnp.float32),
        scratch_types=[
            pltpu.VMEM((b_per_w,), jnp.int32),
            pltpu.VMEM((chunk, cpad), jnp.float32),
            pltpu.SemaphoreType.DMA,
            pltpu.SemaphoreType.DMA,
        ],
    )
    def gather_kernel(table_hbm, idx_hbm, out_hbm, idx_v, rows_v,
                      sem_h, sem_t):
        wid = lax.axis_index("s") * info.num_cores + lax.axis_index("c")
        base = wid * b_per_w
        pltpu.sync_copy(idx_hbm.at[pl.ds(base, b_per_w)], idx_v)
        for ci in range(n_chunks):
            idxc = idx_v.at[pl.ds(ci * chunk, chunk)]
            ch = pltpu.async_copy(
                table_hbm.at[idxc, pl.ds(0, head)],
                rows_v.at[:, pl.ds(0, head)], sem_h)
            ct = pltpu.async_copy(
                table_hbm.at[idxc, pl.ds(head, cpad - head)],
                rows_v.at[:, pl.ds(head, cpad - head)], sem_t)
            ch.wait()
            ct.wait()
            pltpu.sync_copy(rows_v, out_hbm.at[pl.ds(base + ci * chunk, chunk)])

    return gather_kernel(confidence, index)


def _loss_tail_body(idx_ref, ow_ref, os_ref, hd_ref, conf_ref, acc_ref,
                    tail, sem, row_offset=0):
    i = pl.program_id(0)
    ngrid = pl.num_programs(0)
    block_rows = ow_ref.shape[0]
    c = ow_ref.shape[1]
    half = hd_ref.shape[1]
    head = 896
    tw = c - head

    def issue(step, slot):
        base = row_offset + step * block_rows

        def one(j, _):
            jj = j * _NQ
            for k in range(_NQ):
                r = idx_ref[base + jj + k]
                pltpu.make_async_copy(
                    conf_ref.at[pl.ds(r, 1), pl.ds(head, tw)],
                    tail.at[slot, pl.ds(jj + k, 1), :],
                    sem.at[slot, k],
                ).start()
            return 0

        lax.fori_loop(0, block_rows // _NQ, one, 0)

    @pl.when(i == 0)
    def _():
        issue(0, 0)

    @pl.when(i + 1 < ngrid)
    def _():
        issue(i + 1, (i + 1) % 2)

    slot = i % 2
    for k in range(_NQ):
        pltpu.make_async_copy(
            conf_ref.at[pl.ds(0, block_rows // _NQ), pl.ds(head, tw)],
            tail.at[slot, pl.ds(0, block_rows // _NQ), :],
            sem.at[slot, k],
        ).wait()

    ow = ow_ref[...]
    osl = os_ref[...]
    pk = hd_ref[...]
    lo = lax.bitcast_convert_type(lax.shift_left(pk, 16), jnp.float32)
    hi = lax.bitcast_convert_type(
        jnp.bitwise_and(pk, jnp.int32(-65536)), jnp.float32)
    t = jnp.concatenate([lo, hi[:, :head - half], tail[slot]], axis=1)

    mw = jnp.max(ow, axis=1, keepdims=True)
    ew = jnp.exp(ow - mw)
    sw = jnp.sum(ew, axis=1, keepdims=True)
    lse_w = mw + jnp.log(sw)

    ms = jnp.max(osl, axis=1, keepdims=True)
    es = jnp.exp(osl - ms)
    ss = jnp.sum(es, axis=1, keepdims=True)
    lse_s = ms + jnp.log(ss)

    pos = t > 0.0
    safe_t = jnp.where(pos, t, 1.0)
    xlogy = t * jnp.log(safe_t)
    tsum = jnp.sum(t, axis=1, keepdims=True)

    part = (2.0 * jnp.sum(xlogy)
            - jnp.sum(t * ow) - jnp.sum(t * osl)
            + jnp.sum(tsum * (lse_w + lse_s)))

    @pl.when(i == 0)
    def _():
        acc_ref[0, 0] = 0.0

    acc_ref[0, 0] += part

    any_zero = jnp.sum((t == 0.0).astype(jnp.float32)) > 0.0

    @pl.when(any_zero)
    def _():
        pred_w = ew / sw
        pred_s = es / ss
        neg = (t == 0.0).astype(jnp.float32)
        sup = neg * (-jnp.log(jnp.abs(1.0 - pred_w) + 1e-9)
                     - jnp.log(jnp.abs(1.0 - pred_s) + 1e-9))
        acc_ref[0, 0] += jnp.sum(sup)


def _loss_tail_tc(output_w, output_s, head_tgt, confidence, index,
                  block_rows=256, row_offset=0, rows=None):
    b, c = output_w.shape
    if rows is None:
        rows = b
    half = head_tgt.shape[1]
    head = 896
    grid = rows // block_rows
    blk_off = row_offset // block_rows
    grid_spec = pltpu.PrefetchScalarGridSpec(
        num_scalar_prefetch=1,
        grid=(grid,),
        in_specs=[
            pl.BlockSpec((block_rows, c), lambda i, idx: (blk_off + i, 0)),
            pl.BlockSpec((block_rows, c), lambda i, idx: (blk_off + i, 0)),
            pl.BlockSpec((block_rows, half), lambda i, idx: (i, 0)),
            pl.BlockSpec(memory_space=pl.ANY),
        ],
        out_specs=pl.BlockSpec((1, 1), lambda i, idx: (0, 0),
                               memory_space=pltpu.SMEM),
        scratch_shapes=[
            pltpu.VMEM((2, block_rows, c - head), jnp.float32),
            pltpu.SemaphoreType.DMA((2, _NQ)),
        ],
    )
    acc = pl.pallas_call(
        functools.partial(_loss_tail_body, row_offset=row_offset),
        grid_spec=grid_spec,
        out_shape=jax.ShapeDtypeStruct((1, 1), jnp.float32),
    )(index, output_w, output_s, head_tgt, confidence)
    return acc[0, 0]


def _fused_body(idx_ref, ow_ref, os_ref, conf_ref, acc_ref, tgt, sem):
    i = pl.program_id(0)
    ngrid = pl.num_programs(0)
    block_rows = ow_ref.shape[0]
    c = ow_ref.shape[1]

    def issue(step, slot):
        base = step * block_rows

        def one(j, _):
            jj = j * _NQ
            for k in range(_NQ):
                r = idx_ref[base + jj + k]
                pltpu.make_async_copy(
                    conf_ref.at[pl.ds(r, 1), :],
                    tgt.at[slot, pl.ds(jj + k, 1), :],
                    sem.at[slot, k],
                ).start()
            return 0

        lax.fori_loop(0, block_rows // _NQ, one, 0)

    @pl.when(i == 0)
    def _():
        issue(0, 0)

    @pl.when(i + 1 < ngrid)
    def _():
        issue(i + 1, (i + 1) % 2)

    slot = i % 2
    for k in range(_NQ):
        pltpu.make_async_copy(
            conf_ref.at[pl.ds(0, block_rows // _NQ), :],
            tgt.at[slot, pl.ds(0, block_rows // _NQ), :],
            sem.at[slot, k],
        ).wait()

    ow = ow_ref[...]
    osl = os_ref[...]
    t = tgt[slot]

    mw = jnp.max(ow, axis=1, keepdims=True)
    ew = jnp.exp(ow - mw)
    sw = jnp.sum(ew, axis=1, keepdims=True)
    lse_w = mw + jnp.log(sw)

    ms = jnp.max(osl, axis=1, keepdims=True)
    es = jnp.exp(osl - ms)
    ss = jnp.sum(es, axis=1, keepdims=True)
    lse_s = ms + jnp.log(ss)

    pos = t > 0.0
    safe_t = jnp.where(pos, t, 1.0)
    xlogy = t * jnp.log(safe_t)
    tsum = jnp.sum(t, axis=1, keepdims=True)

    part = (2.0 * jnp.sum(xlogy)
            - jnp.sum(t * ow) - jnp.sum(t * osl)
            + jnp.sum(tsum * (lse_w + lse_s)))

    @pl.when(i == 0)
    def _():
        acc_ref[0, 0] = 0.0

    acc_ref[0, 0] += part

    any_zero = jnp.sum((t == 0.0).astype(jnp.float32)) > 0.0

    @pl.when(any_zero)
    def _():
        pred_w = ew / sw
        pred_s = es / ss
        neg = (t == 0.0).astype(jnp.float32)
        sup = neg * (-jnp.log(jnp.abs(1.0 - pred_w) + 1e-9)
                     - jnp.log(jnp.abs(1.0 - pred_s) + 1e-9))
        acc_ref[0, 0] += jnp.sum(sup)


def _fused_tc(output_w, output_s, confidence, index, block_rows=256):
    b, c = output_w.shape
    grid = b // block_rows
    grid_spec = pltpu.PrefetchScalarGridSpec(
        num_scalar_prefetch=1,
        grid=(grid,),
        in_specs=[
            pl.BlockSpec((block_rows, c), lambda i, idx: (i, 0)),
            pl.BlockSpec((block_rows, c), lambda i, idx: (i, 0)),
            pl.BlockSpec(memory_space=pl.ANY),
        ],
        out_specs=pl.BlockSpec((1, 1), lambda i, idx: (0, 0),
                               memory_space=pltpu.SMEM),
        scratch_shapes=[
            pltpu.VMEM((2, block_rows, c), jnp.float32),
            pltpu.SemaphoreType.DMA((2, _NQ)),
        ],
    )
    acc = pl.pallas_call(
        _fused_body,
        grid_spec=grid_spec,
        out_shape=jax.ShapeDtypeStruct((1, 1), jnp.float32),
    )(index, output_w, output_s, confidence)
    return acc[0, 0] / b


def kernel(output_w, output_s, feat_w, feat_s, confidence, index):
    del feat_w, feat_s
    b = index.shape[0]
    head_tgt = _sc_gather_head_bf16(confidence, index)
    acc = _loss_tail_tc(output_w, output_s, head_tgt, confidence, index,
                        block_rows=512)
    return acc / b


# speedup vs baseline: 1.0250x; 1.0007x over previous
import functools

import jax
import jax.numpy as jnp
from jax import lax
from jax.experimental import pallas as pl
from jax.experimental.pallas import tpu as pltpu
from jax.experimental.pallas import tpu_sc as plsc


def _sc_gather(confidence, index):
    n_rows, n_cols = confidence.shape
    b = index.shape[0]
    info = plsc.get_sparse_core_info()
    nw = info.num_cores * info.num_subcores
    b_per_w = b // nw
    mesh = plsc.VectorSubcoreMesh(core_axis_name="c", subcore_axis_name="s")

    @functools.partial(
        pl.kernel,
        mesh=mesh,
        out_type=jax.ShapeDtypeStruct((b, n_cols), jnp.float32),
        scratch_types=[
            pltpu.VMEM((b_per_w,), jnp.int32),
            pltpu.VMEM((b_per_w, n_cols), jnp.float32),
            pltpu.SemaphoreType.DMA,
        ],
        compiler_params=pltpu.CompilerParams(use_tc_tiling_on_sc=False),
    )
    def gather_kernel(table_hbm, idx_hbm, out_hbm, idx_v, rows_v, sem):
        wid = lax.axis_index("s") * info.num_cores + lax.axis_index("c")
        base = wid * b_per_w
        pltpu.sync_copy(idx_hbm.at[pl.ds(base, b_per_w)], idx_v)
        pltpu.async_copy(table_hbm.at[idx_v], rows_v, sem).wait()
        pltpu.sync_copy(rows_v, out_hbm.at[pl.ds(base, b_per_w)])

    return gather_kernel(confidence, index)


def _loss_body(ow_ref, os_ref, tg_ref, acc_ref):
    i = pl.program_id(0)
    ow = ow_ref[...]
    osl = os_ref[...]
    c = ow_ref.shape[1]
    tg = tg_ref[...]
    t = tg[:, :c]

    mw = jnp.max(ow, axis=1, keepdims=True)
    ew = jnp.exp(ow - mw)
    sw = jnp.sum(ew, axis=1, keepdims=True)
    lse_w = mw + jnp.log(sw)

    ms = jnp.max(osl, axis=1, keepdims=True)
    es = jnp.exp(osl - ms)
    ss = jnp.sum(es, axis=1, keepdims=True)
    lse_s = ms + jnp.log(ss)

    pos = t > 0.0
    safe_t = jnp.where(pos, t, 1.0)
    xlogy = t * jnp.log(safe_t)
    tsum = jnp.sum(t, axis=1, keepdims=True)

    part = (2.0 * jnp.sum(xlogy)
            - jnp.sum(t * ow) - jnp.sum(t * osl)
            + jnp.sum(tsum * (lse_w + lse_s)))

    @pl.when(i == 0)
    def _():
        acc_ref[0, 0] = 0.0

    acc_ref[0, 0] += part

    any_zero = jnp.sum((t == 0.0).astype(jnp.float32)) > 0.0

    @pl.when(any_zero)
    def _():
        pred_w = ew / sw
        pred_s = es / ss
        neg = (t == 0.0).astype(jnp.float32)
        sup = neg * (-jnp.log(jnp.abs(1.0 - pred_w) + 1e-9)
                     - jnp.log(jnp.abs(1.0 - pred_s) + 1e-9))
        acc_ref[0, 0] += jnp.sum(sup)


def _loss_tc(output_w, output_s, target, block_rows=256, interpret=False):
    b, c = output_w.shape
    ct = target.shape[1]
    grid = b // block_rows
    acc = pl.pallas_call(
        _loss_body,
        grid=(grid,),
        in_specs=[
            pl.BlockSpec((block_rows, c), lambda i: (i, 0)),
            pl.BlockSpec((block_rows, c), lambda i: (i, 0)),
            pl.BlockSpec((block_rows, ct), lambda i: (i, 0)),
        ],
        out_specs=pl.BlockSpec((1, 1), lambda i: (0, 0),
                               memory_space=pltpu.SMEM),
        out_shape=jax.ShapeDtypeStruct((1, 1), jnp.float32),
        interpret=interpret,
    )(output_w, output_s, target)
    return acc[0, 0] / b


_NQ = 8


def _sc_gather_head(confidence, index, head=896):
    b = index.shape[0]
    info = plsc.get_sparse_core_info()
    nw = info.num_cores * info.num_subcores
    b_per_w = b // nw
    mesh = plsc.VectorSubcoreMesh(core_axis_name="c", subcore_axis_name="s")

    @functools.partial(
        pl.kernel,
        mesh=mesh,
        out_type=jax.ShapeDtypeStruct((b, head), jnp.float32),
        scratch_types=[
            pltpu.VMEM((b_per_w,), jnp.int32),
            pltpu.VMEM((b_per_w, head), jnp.float32),
            pltpu.SemaphoreType.DMA,
        ],
    )
    def gather_kernel(table_hbm, idx_hbm, out_hbm, idx_v, rows_v, sem):
        wid = lax.axis_index("s") * info.num_cores + lax.axis_index("c")
        base = wid * b_per_w
        pltpu.sync_copy(idx_hbm.at[pl.ds(base, b_per_w)], idx_v)
        pltpu.async_copy(
            table_hbm.at[idx_v, pl.ds(0, head)], rows_v, sem).wait()
        pltpu.sync_copy(rows_v, out_hbm.at[pl.ds(base, b_per_w)])

    return gather_kernel(confidence, index)


def _rne_bf16_hi(x_i32):
    rnd = jnp.bitwise_and(lax.shift_right_logical(x_i32, 16), 1) + 0x7FFF
    return jnp.bitwise_and(x_i32 + rnd, jnp.int32(-65536))


def _sc_gather_head_bf16(confidence, index, head=896):
    b = index.shape[0]
    half = 512
    info = plsc.get_sparse_core_info()
    nw = info.num_cores * info.num_subcores
    b_per_w = b // nw
    mesh = plsc.VectorSubcoreMesh(core_axis_name="c", subcore_axis_name="s")
    lanes = info.num_lanes

    chunk = 32
    n_chunks = b_per_w // chunk

    @functools.partial(
        pl.kernel,
        mesh=mesh,
        out_type=jax.ShapeDtypeStruct((b, half), jnp.int32),
        scratch_types=[
            pltpu.VMEM((b_per_w,), jnp.int32),
            pltpu.VMEM((chunk, head), jnp.float32),
            pltpu.VMEM((chunk, head), jnp.float32),
            pltpu.VMEM((chunk, half), jnp.int32),
            pltpu.VMEM((chunk, half), jnp.int32),
            pltpu.SemaphoreType.DMA,
            pltpu.SemaphoreType.DMA,
        ],
        compiler_params=pltpu.CompilerParams(needs_layout_passes=False),
    )
    def gather_kernel(table_hbm, idx_hbm, out_hbm, idx_v, rows_a, rows_b,
                      pk_a, pk_b, sem_a, sem_b):
        wid = lax.axis_index("s") * info.num_cores + lax.axis_index("c")
        base = wid * b_per_w
        pltpu.sync_copy(idx_hbm.at[pl.ds(base, b_per_w)], idx_v)
        bufs = [(rows_a, pk_a, sem_a), (rows_b, pk_b, sem_b)]

        def start(ci):
            rows, _, sem = bufs[ci % 2]
            idxc = idx_v.at[pl.ds(ci * chunk, chunk)]
            return pltpu.async_copy(
                table_hbm.at[idxc, pl.ds(0, head)], rows, sem)

        handles = [None] * n_chunks
        handles[0] = start(0)
        for ci in range(n_chunks):
            rows_v, pk_v, _ = bufs[ci % 2]
            handles[ci].wait()
            if ci + 1 < n_chunks:
                handles[ci + 1] = start(ci + 1)

            def row(r, _):
                for j in range(half // lanes):
                    lo_f = rows_v[r, pl.ds(j * lanes, lanes)]
                    if (j + 1) * lanes <= head - half:
                        hi_f = rows_v[r, pl.ds(half + j * lanes, lanes)]
                    else:
                        hi_f = jnp.zeros((lanes,), jnp.float32)
                    pk = plsc.pack(lo_f, hi_f,
                                   format=plsc.PackFormat.INTERLEAVED)
                    pk_v[r, pl.ds(j * lanes, lanes)] = plsc.bitcast(
                        pk, jnp.int32)
                return 0

            lax.fori_loop(0, chunk, row, 0)
            pltpu.sync_copy(
                pk_v, out_hbm.at[pl.ds(base + ci * chunk, chunk)])

    return gather_kernel(confidence, index)


def _sc_gather_split(confidence, index):
    n_rows, n_cols = confidence.shape
    head = 896
    cpad = 1024
    b = index.shape[0]
    info = plsc.get_sparse_core_info()
    nw = info.num_cores * info.num_subcores
    b_per_w = b // nw
    chunk = 64
    n_chunks = b_per_w // chunk
    mesh = plsc.VectorSubcoreMesh(core_axis_name="c", subcore_axis_name="s")

    @functools.partial(
        pl.kernel,
        mesh=mesh,
        out_type=jax.ShapeDtypeStruct((b, cpad), jnp.float32),
        scratch_types=[
            pltpu.VMEM((b_per_w,), jnp.int32),
            pltpu.VMEM((chunk, cpad), jnp.float32),
            pltpu.SemaphoreType.DMA,
            pltpu.SemaphoreType.DMA,
        ],
    )
    def gather_kernel(table_hbm, idx_hbm, out_hbm, idx_v, rows_v,
                      sem_h, sem_t):
        wid = lax.axis_index("s") * info.num_cores + lax.axis_index("c")
        base = wid * b_per_w
        pltpu.sync_copy(idx_hbm.at[pl.ds(base, b_per_w)], idx_v)
        for ci in range(n_chunks):
            idxc = idx_v.at[pl.ds(ci * chunk, chunk)]
            ch = pltpu.async_copy(
                table_hbm.at[idxc, pl.ds(0, head)],
                rows_v.at[:, pl.ds(0, head)], sem_h)
            ct = pltpu.async_copy(
                table_hbm.at[idxc, pl.ds(head, cpad - head)],
                rows_v.at[:, pl.ds(head, cpad - head)], sem_t)
            ch.wait()
            ct.wait()
            pltpu.sync_copy(rows_v, out_hbm.at[pl.ds(base + ci * chunk, chunk)])

    return gather_kernel(confidence, index)


def _loss_tail_body(idx_ref, ow_ref, os_ref, hd_ref, conf_ref, acc_ref,
                    tail, sem, row_offset=0):
    i = pl.program_id(0)
    ngrid = pl.num_programs(0)
    block_rows = ow_ref.shape[0]
    c = ow_ref.shape[1]
    half = hd_ref.shape[1]
    head = 896
    tw = c - head

    def issue(step, slot):
        base = row_offset + step * block_rows

        def one(j, _):
            jj = j * _NQ
            for k in range(_NQ):
                r = idx_ref[base + jj + k]
                pltpu.make_async_copy(
                    conf_ref.at[pl.ds(r, 1), pl.ds(head, tw)],
                    tail.at[slot, pl.ds(jj + k, 1), :],
                    sem.at[slot, k],
                ).start()
            return 0

        lax.fori_loop(0, block_rows // _NQ, one, 0)

    @pl.when(i == 0)
    def _():
        issue(0, 0)

    @pl.when(i + 1 < ngrid)
    def _():
        issue(i + 1, (i + 1) % 2)

    slot = i % 2
    for k in range(_NQ):
        pltpu.make_async_copy(
            conf_ref.at[pl.ds(0, block_rows // _NQ), pl.ds(head, tw)],
            tail.at[slot, pl.ds(0, block_rows // _NQ), :],
            sem.at[slot, k],
        ).wait()

    ow = ow_ref[...]
    osl = os_ref[...]
    pk = hd_ref[...]
    lo = lax.bitcast_convert_type(lax.shift_left(pk, 16), jnp.float32)
    hi = lax.bitcast_convert_type(
        jnp.bitwise_and(pk, jnp.int32(-65536)), jnp.float32)
    t = jnp.concatenate([lo, hi[:, :head - half], tail[slot]], axis=1)

    mw = jnp.max(ow, axis=1, keepdims=True)
    ew = jnp.exp(ow - mw)
    sw = jnp.sum(ew, axis=1, keepdims=True)
    lse_w = mw + jnp.log(sw)

    ms = jnp.max(osl, axis=1, keepdims=True)
    es = jnp.exp(osl - ms)
    ss = jnp.sum(es, axis=1, keepdims=True)
    lse_s = ms + jnp.log(ss)

    pos = t > 0.0
    safe_t = jnp.where(pos, t, 1.0)
    xlogy = t * jnp.log(safe_t)
    tsum = jnp.sum(t, axis=1, keepdims=True)

    part = (2.0 * jnp.sum(xlogy)
            - jnp.sum(t * ow) - jnp.sum(t * osl)
            + jnp.sum(tsum * (lse_w + lse_s)))

    @pl.when(i == 0)
    def _():
        acc_ref[0, 0] = 0.0

    acc_ref[0, 0] += part

    any_zero = jnp.sum((t == 0.0).astype(jnp.float32)) > 0.0

    @pl.when(any_zero)
    def _():
        pred_w = ew / sw
        pred_s = es / ss
        neg = (t == 0.0).astype(jnp.float32)
        sup = neg * (-jnp.log(jnp.abs(1.0 - pred_w) + 1e-9)
                     - jnp.log(jnp.abs(1.0 - pred_s) + 1e-9))
        acc_ref[0, 0] += jnp.sum(sup)


def _loss_tail_tc(output_w, output_s, head_tgt, confidence, index,
                  block_rows=256, row_offset=0, rows=None):
    b, c = output_w.shape
    if rows is None:
        rows = b
    half = head_tgt.shape[1]
    head = 896
    grid = rows // block_rows
    blk_off = row_offset // block_rows
    grid_spec = pltpu.PrefetchScalarGridSpec(
        num_scalar_prefetch=1,
        grid=(grid,),
        in_specs=[
            pl.BlockSpec((block_rows, c), lambda i, idx: (blk_off + i, 0)),
            pl.BlockSpec((block_rows, c), lambda i, idx: (blk_off + i, 0)),
            pl.BlockSpec((block_rows, half), lambda i, idx: (i, 0)),
            pl.BlockSpec(memory_space=pl.ANY),
        ],
        out_specs=pl.BlockSpec((1, 1), lambda i, idx: (0, 0),
                               memory_space=pltpu.SMEM),
        scratch_shapes=[
            pltpu.VMEM((2, block_rows, c - head), jnp.float32),
            pltpu.SemaphoreType.DMA((2, _NQ)),
        ],
    )
    acc = pl.pallas_call(
        functools.partial(_loss_tail_body, row_offset=row_offset),
        grid_spec=grid_spec,
        out_shape=jax.ShapeDtypeStruct((1, 1), jnp.float32),
    )(index, output_w, output_s, head_tgt, confidence)
    return acc[0, 0]


def _fused_body(idx_ref, ow_ref, os_ref, conf_ref, acc_ref, tgt, sem):
    i = pl.program_id(0)
    ngrid = pl.num_programs(0)
    block_rows = ow_ref.shape[0]
    c = ow_ref.shape[1]

    def issue(step, slot):
        base = step * block_rows

        def one(j, _):
            jj = j * _NQ
            for k in range(_NQ):
                r = idx_ref[base + jj + k]
                pltpu.make_async_copy(
                    conf_ref.at[pl.ds(r, 1), :],
                    tgt.at[slot, pl.ds(jj + k, 1), :],
                    sem.at[slot, k],
                ).start()
            return 0

        lax.fori_loop(0, block_rows // _NQ, one, 0)

    @pl.when(i == 0)
    def _():
        issue(0, 0)

    @pl.when(i + 1 < ngrid)
    def _():
        issue(i + 1, (i + 1) % 2)

    slot = i % 2
    for k in range(_NQ):
        pltpu.make_async_copy(
            conf_ref.at[pl.ds(0, block_rows // _NQ), :],
            tgt.at[slot, pl.ds(0, block_rows // _NQ), :],
            sem.at[slot, k],
        ).wait()

    ow = ow_ref[...]
    osl = os_ref[...]
    t = tgt[slot]

    mw = jnp.max(ow, axis=1, keepdims=True)
    ew = jnp.exp(ow - mw)
    sw = jnp.sum(ew, axis=1, keepdims=True)
    lse_w = mw + jnp.log(sw)

    ms = jnp.max(osl, axis=1, keepdims=True)
    es = jnp.exp(osl - ms)
    ss = jnp.sum(es, axis=1, keepdims=True)
    lse_s = ms + jnp.log(ss)

    pos = t > 0.0
    safe_t = jnp.where(pos, t, 1.0)
    xlogy = t * jnp.log(safe_t)
    tsum = jnp.sum(t, axis=1, keepdims=True)

    part = (2.0 * jnp.sum(xlogy)
            - jnp.sum(t * ow) - jnp.sum(t * osl)
            + jnp.sum(tsum * (lse_w + lse_s)))

    @pl.when(i == 0)
    def _():
        acc_ref[0, 0] = 0.0

    acc_ref[0, 0] += part

    any_zero = jnp.sum((t == 0.0).astype(jnp.float32)) > 0.0

    @pl.when(any_zero)
    def _():
        pred_w = ew / sw
        pred_s = es / ss
        neg = (t == 0.0).astype(jnp.float32)
        sup = neg * (-jnp.log(jnp.abs(1.0 - pred_w) + 1e-9)
                     - jnp.log(jnp.abs(1.0 - pred_s) + 1e-9))
        acc_ref[0, 0] += jnp.sum(sup)


def _fused_tc(output_w, output_s, confidence, index, block_rows=256):
    b, c = output_w.shape
    grid = b // block_rows
    grid_spec = pltpu.PrefetchScalarGridSpec(
        num_scalar_prefetch=1,
        grid=(grid,),
        in_specs=[
            pl.BlockSpec((block_rows, c), lambda i, idx: (i, 0)),
            pl.BlockSpec((block_rows, c), lambda i, idx: (i, 0)),
            pl.BlockSpec(memory_space=pl.ANY),
        ],
        out_specs=pl.BlockSpec((1, 1), lambda i, idx: (0, 0),
                               memory_space=pltpu.SMEM),
        scratch_shapes=[
            pltpu.VMEM((2, block_rows, c), jnp.float32),
            pltpu.SemaphoreType.DMA((2, _NQ)),
        ],
    )
    acc = pl.pallas_call(
        _fused_body,
        grid_spec=grid_spec,
        out_shape=jax.ShapeDtypeStruct((1, 1), jnp.float32),
    )(index, output_w, output_s, confidence)
    return acc[0, 0] / b


def kernel(output_w, output_s, feat_w, feat_s, confidence, index):
    del feat_w, feat_s
    b = index.shape[0]
    head_tgt = _sc_gather_head_bf16(confidence, index)
    acc = _loss_tail_tc(output_w, output_s, head_tgt, confidence, index,
                        block_rows=512)
    return acc / b


# speedup vs baseline: 1.0254x; 1.0005x over previous
import functools

import jax
import jax.numpy as jnp
from jax import lax
from jax.experimental import pallas as pl
from jax.experimental.pallas import tpu as pltpu
from jax.experimental.pallas import tpu_sc as plsc

_NQ = 8


def _sc_gather_head_bf16(confidence, index, head=896):
    b = index.shape[0]
    half = 512
    info = plsc.get_sparse_core_info()
    nw = info.num_cores * info.num_subcores
    b_per_w = b // nw
    lanes = info.num_lanes
    mesh = plsc.VectorSubcoreMesh(core_axis_name="c", subcore_axis_name="s")
    chunk = 32
    n_chunks = b_per_w // chunk

    @functools.partial(
        pl.kernel,
        mesh=mesh,
        out_type=jax.ShapeDtypeStruct((b, half), jnp.int32),
        scratch_types=[
            pltpu.VMEM((b_per_w,), jnp.int32),
            pltpu.VMEM((chunk, head), jnp.float32),
            pltpu.VMEM((chunk, head), jnp.float32),
            pltpu.VMEM((chunk, half), jnp.int32),
            pltpu.VMEM((chunk, half), jnp.int32),
            pltpu.SemaphoreType.DMA,
            pltpu.SemaphoreType.DMA,
        ],
        compiler_params=pltpu.CompilerParams(needs_layout_passes=False),
    )
    def gather_kernel(table_hbm, idx_hbm, out_hbm, idx_v, rows_a, rows_b,
                      pk_a, pk_b, sem_a, sem_b):
        wid = lax.axis_index("s") * info.num_cores + lax.axis_index("c")
        base = wid * b_per_w
        pltpu.sync_copy(idx_hbm.at[pl.ds(base, b_per_w)], idx_v)
        bufs = [(rows_a, pk_a, sem_a), (rows_b, pk_b, sem_b)]

        def start(ci):
            rows, _, sem = bufs[ci % 2]
            idxc = idx_v.at[pl.ds(ci * chunk, chunk)]
            return pltpu.async_copy(
                table_hbm.at[idxc, pl.ds(0, head)], rows, sem)

        handles = [None] * n_chunks
        handles[0] = start(0)
        for ci in range(n_chunks):
            rows_v, pk_v, _ = bufs[ci % 2]
            handles[ci].wait()
            if ci + 1 < n_chunks:
                handles[ci + 1] = start(ci + 1)

            def row(r, _):
                for j in range(half // lanes):
                    lo_f = rows_v[r, pl.ds(j * lanes, lanes)]
                    if (j + 1) * lanes <= head - half:
                        hi_f = rows_v[r, pl.ds(half + j * lanes, lanes)]
                    else:
                        hi_f = jnp.zeros((lanes,), jnp.float32)
                    pk = plsc.pack(lo_f, hi_f,
                                   format=plsc.PackFormat.INTERLEAVED)
                    pk_v[r, pl.ds(j * lanes, lanes)] = plsc.bitcast(
                        pk, jnp.int32)
                return 0

            lax.fori_loop(0, chunk, row, 0)
            pltpu.sync_copy(
                pk_v, out_hbm.at[pl.ds(base + ci * chunk, chunk)])

    return gather_kernel(confidence, index)


def _loss_tail_body(idx_ref, ow_ref, os_ref, hd_ref, conf_ref, acc_ref,
                    tail, sem):
    i = pl.program_id(0)
    ngrid = pl.num_programs(0)
    block_rows = ow_ref.shape[0]
    c = ow_ref.shape[1]
    half = hd_ref.shape[1]
    head = 896
    tw = c - head

    def issue(step, slot):
        base = step * block_rows

        def one(j, _):
            jj = j * _NQ
            for k in range(_NQ):
                r = idx_ref[base + jj + k]
                pltpu.make_async_copy(
                    conf_ref.at[pl.ds(r, 1), pl.ds(head, tw)],
                    tail.at[slot, pl.ds(jj + k, 1), :],
                    sem.at[slot, k],
                ).start()
            return 0

        lax.fori_loop(0, block_rows // _NQ, one, 0)

    @pl.when(i == 0)
    def _():
        issue(0, 0)

    @pl.when(i + 1 < ngrid)
    def _():
        issue(i + 1, (i + 1) % 2)

    slot = i % 2
    for k in range(_NQ):
        pltpu.make_async_copy(
            conf_ref.at[pl.ds(0, block_rows // _NQ), pl.ds(head, tw)],
            tail.at[slot, pl.ds(0, block_rows // _NQ), :],
            sem.at[slot, k],
        ).wait()

    ow = ow_ref[...]
    osl = os_ref[...]
    pk = hd_ref[...]
    lo = lax.bitcast_convert_type(lax.shift_left(pk, 16), jnp.float32)
    hi = lax.bitcast_convert_type(
        jnp.bitwise_and(pk, jnp.int32(-65536)), jnp.float32)
    t = jnp.concatenate([lo, hi[:, :head - half], tail[slot]], axis=1)

    mw = jnp.max(ow, axis=1, keepdims=True)
    ew = jnp.exp(ow - mw)
    sw = jnp.sum(ew, axis=1, keepdims=True)
    lse_w = mw + jnp.log(sw)

    ms = jnp.max(osl, axis=1, keepdims=True)
    es = jnp.exp(osl - ms)
    ss = jnp.sum(es, axis=1, keepdims=True)
    lse_s = ms + jnp.log(ss)

    pos = t > 0.0
    safe_t = jnp.where(pos, t, 1.0)
    xlogy = t * jnp.log(safe_t)
    tsum = jnp.sum(t, axis=1, keepdims=True)

    part = (2.0 * jnp.sum(xlogy)
            - jnp.sum(t * ow) - jnp.sum(t * osl)
            + jnp.sum(tsum * (lse_w + lse_s)))

    @pl.when(i == 0)
    def _():
        acc_ref[0, 0] = 0.0

    acc_ref[0, 0] += part

    any_zero = jnp.sum((t == 0.0).astype(jnp.float32)) > 0.0

    @pl.when(any_zero)
    def _():
        pred_w = ew / sw
        pred_s = es / ss
        neg = (t == 0.0).astype(jnp.float32)
        sup = neg * (-jnp.log(jnp.abs(1.0 - pred_w) + 1e-9)
                     - jnp.log(jnp.abs(1.0 - pred_s) + 1e-9))
        acc_ref[0, 0] += jnp.sum(sup)


def _loss_tail_tc(output_w, output_s, head_tgt, confidence, index,
                  block_rows=512):
    b, c = output_w.shape
    half = head_tgt.shape[1]
    head = 896
    grid = b // block_rows
    grid_spec = pltpu.PrefetchScalarGridSpec(
        num_scalar_prefetch=1,
        grid=(grid,),
        in_specs=[
            pl.BlockSpec((block_rows, c), lambda i, idx: (i, 0)),
            pl.BlockSpec((block_rows, c), lambda i, idx: (i, 0)),
            pl.BlockSpec((block_rows, half), lambda i, idx: (i, 0)),
            pl.BlockSpec(memory_space=pl.ANY),
        ],
        out_specs=pl.BlockSpec((1, 1), lambda i, idx: (0, 0),
                               memory_space=pltpu.SMEM),
        scratch_shapes=[
            pltpu.VMEM((2, block_rows, c - head), jnp.float32),
            pltpu.SemaphoreType.DMA((2, _NQ)),
        ],
    )
    acc = pl.pallas_call(
        _loss_tail_body,
        grid_spec=grid_spec,
        out_shape=jax.ShapeDtypeStruct((1, 1), jnp.float32),
    )(index, output_w, output_s, head_tgt, confidence)
    return acc[0, 0] / b


def kernel(output_w, output_s, feat_w, feat_s, confidence, index):
    del feat_w, feat_s
    head_tgt = _sc_gather_head_bf16(confidence, index)
    return _loss_tail_tc(output_w, output_s, head_tgt, confidence, index)
